# initial kernel scaffold (unmeasured)
import jax
import jax.numpy as jnp
from jax import lax
from jax.experimental import pallas as pl
from jax.experimental.pallas import tpu as pltpu


def _allreduce_x_body(
    p_ref, out_ref, acc, recv, send_sem1, recv_sem1, send_sem2, recv_sem2,
    copy_sem1, copy_sem2,
):
    half, n = acc.shape
    my_x = lax.axis_index("x")
    my_y = lax.axis_index("y")
    my_z = lax.axis_index("z")
    x_peer = (1 - my_x, my_y, my_z)
    y_peer = (my_x, 1 - my_y, my_z)

    barrier = pltpu.get_barrier_semaphore()
    for nbr in (x_peer, y_peer):
        pl.semaphore_signal(
            barrier, inc=1, device_id=nbr, device_id_type=pl.DeviceIdType.MESH
        )
    pl.semaphore_wait(barrier, 2)

    row0 = my_y * half

    rdma1 = pltpu.make_async_remote_copy(
        src_ref=p_ref.at[pl.ds(row0, half), :],
        dst_ref=recv,
        send_sem=send_sem1,
        recv_sem=recv_sem1,
        device_id=x_peer,
        device_id_type=pl.DeviceIdType.MESH,
    )
    rdma1.start()
    copy1 = pltpu.make_async_copy(p_ref.at[pl.ds(row0, half), :], acc, copy_sem1)
    copy1.start()
    copy1.wait()
    rdma1.wait()
    acc[...] = acc[...] + recv[...]

    rdma2 = pltpu.make_async_remote_copy(
        src_ref=acc,
        dst_ref=out_ref.at[pl.ds(row0, half), :],
        send_sem=send_sem2,
        recv_sem=recv_sem2,
        device_id=y_peer,
        device_id_type=pl.DeviceIdType.MESH,
    )
    rdma2.start()
    copy2 = pltpu.make_async_copy(acc, out_ref.at[pl.ds(row0, half), :], copy_sem2)
    copy2.start()
    copy2.wait()
    rdma2.wait()


def _allreduce_x(partial):
    m, n = partial.shape
    half = m // 2
    return pl.pallas_call(
        _allreduce_x_body,
        out_shape=jax.ShapeDtypeStruct((m, n), partial.dtype),
        in_specs=[pl.BlockSpec(memory_space=pltpu.ANY)],
        out_specs=pl.BlockSpec(memory_space=pltpu.ANY),
        scratch_shapes=[
            pltpu.VMEM((half, n), partial.dtype),
            pltpu.VMEM((half, n), partial.dtype),
            pltpu.SemaphoreType.DMA,
            pltpu.SemaphoreType.DMA,
            pltpu.SemaphoreType.DMA,
            pltpu.SemaphoreType.DMA,
            pltpu.SemaphoreType.DMA,
            pltpu.SemaphoreType.DMA,
        ],
        compiler_params=pltpu.CompilerParams(collective_id=0),
    )(partial)


def kernel(dy, W):
    partial = lax.dot_general(
        dy, W,
        dimension_numbers=(((1,), (1,)), ((), ())),
        preferred_element_type=jnp.float32,
    )
    return _allreduce_x(partial)


# baseline (device time: 1156695 ns/iter reference)
import jax
import jax.numpy as jnp
from jax import lax
from jax.experimental import pallas as pl
from jax.experimental.pallas import tpu as pltpu

NCHUNK = 4


def _allreduce_x_body(
    p_ref, out_ref, acc, recv, send_sem1, recv_sem1, send_sem2, recv_sem2,
    copy_sem,
):
    _, rows, n = recv.shape
    half = NCHUNK * rows
    my_x = lax.axis_index("x")
    my_y = lax.axis_index("y")
    my_z = lax.axis_index("z")
    x_peer = (1 - my_x, my_y, my_z)
    y_peer = (my_x, 1 - my_y, my_z)

    barrier = pltpu.get_barrier_semaphore()
    for nbr in (x_peer, y_peer):
        pl.semaphore_signal(
            barrier, inc=1, device_id=nbr, device_id_type=pl.DeviceIdType.MESH
        )
    pl.semaphore_wait(barrier, 2)

    row0 = my_y * half

    for i in range(NCHUNK):
        slot = i % 2
        r0 = row0 + i * rows

        rdma1 = pltpu.make_async_remote_copy(
            src_ref=p_ref.at[pl.ds(r0, rows), :],
            dst_ref=recv.at[slot],
            send_sem=send_sem1.at[slot],
            recv_sem=recv_sem1.at[slot],
            device_id=x_peer,
            device_id_type=pl.DeviceIdType.MESH,
        )
        rdma1.start()
        copy1 = pltpu.make_async_copy(
            p_ref.at[pl.ds(r0, rows), :], acc, copy_sem
        )
        copy1.start()
        copy1.wait()
        rdma1.wait()
        acc[...] = acc[...] + recv[slot]

        rdma2 = pltpu.make_async_remote_copy(
            src_ref=acc,
            dst_ref=out_ref.at[pl.ds(r0, rows), :],
            send_sem=send_sem2.at[slot],
            recv_sem=recv_sem2.at[slot],
            device_id=y_peer,
            device_id_type=pl.DeviceIdType.MESH,
        )
        rdma2.start()
        copy2 = pltpu.make_async_copy(
            acc, out_ref.at[pl.ds(r0, rows), :], copy_sem
        )
        copy2.start()
        copy2.wait()
        rdma2.wait()


def _allreduce_x(partial):
    m, n = partial.shape
    rows = m // 2 // NCHUNK
    return pl.pallas_call(
        _allreduce_x_body,
        out_shape=jax.ShapeDtypeStruct((m, n), partial.dtype),
        in_specs=[pl.BlockSpec(memory_space=pl.ANY)],
        out_specs=pl.BlockSpec(memory_space=pl.ANY),
        scratch_shapes=[
            pltpu.VMEM((rows, n), partial.dtype),
            pltpu.VMEM((2, rows, n), partial.dtype),
            pltpu.SemaphoreType.DMA((2,)),
            pltpu.SemaphoreType.DMA((2,)),
            pltpu.SemaphoreType.DMA((2,)),
            pltpu.SemaphoreType.DMA((2,)),
            pltpu.SemaphoreType.DMA,
        ],
        compiler_params=pltpu.CompilerParams(collective_id=0),
    )(partial)


def kernel(dy, W):
    partial = lax.dot_general(
        dy, W,
        dimension_numbers=(((1,), (1,)), ((), ())),
        preferred_element_type=jnp.float32,
    )
    return _allreduce_x(partial)


# device time: 698639 ns/iter; 1.6556x vs baseline; 1.6556x over previous
import jax
import jax.numpy as jnp
from jax import lax
from jax.experimental import pallas as pl
from jax.experimental.pallas import tpu as pltpu

NCHUNK = 8
NSLOT = 4


def _allreduce_x_body(
    p_ref, out_ref, acc, recv, send_sem1, recv_sem1, send_sem2, recv_sem2,
    copy1_sem, copy2_sem,
):
    _, rows, n = recv.shape
    half = NCHUNK * rows
    my_x = lax.axis_index("x")
    my_y = lax.axis_index("y")
    my_z = lax.axis_index("z")
    x_peer = (1 - my_x, my_y, my_z)
    y_peer = (my_x, 1 - my_y, my_z)

    barrier = pltpu.get_barrier_semaphore()
    for nbr in (x_peer, y_peer):
        pl.semaphore_signal(
            barrier, inc=1, device_id=nbr, device_id_type=pl.DeviceIdType.MESH
        )
    pl.semaphore_wait(barrier, 2)

    out0 = my_y * half

    def rdma1(i):
        return pltpu.make_async_remote_copy(
            src_ref=p_ref.at[pl.ds(i * rows, rows), :],
            dst_ref=recv.at[i % NSLOT],
            send_sem=send_sem1.at[i % NSLOT],
            recv_sem=recv_sem1.at[i % NSLOT],
            device_id=x_peer,
            device_id_type=pl.DeviceIdType.MESH,
        )

    def rdma2(i):
        return pltpu.make_async_remote_copy(
            src_ref=acc.at[i % 2],
            dst_ref=out_ref.at[pl.ds(out0 + i * rows, rows), :],
            send_sem=send_sem2.at[i % NSLOT],
            recv_sem=recv_sem2.at[i % NSLOT],
            device_id=y_peer,
            device_id_type=pl.DeviceIdType.MESH,
        )

    rdma1(0).start()
    for i in range(NCHUNK):
        s = i % 2
        if i + 1 < NCHUNK:
            rdma1(i + 1).start()
        if i >= 2:
            rdma2(i - 2).wait_send()
            pltpu.make_async_copy(
                acc.at[s], out_ref.at[pl.ds(out0 + (i - 2) * rows, rows), :],
                copy2_sem.at[s],
            ).wait()
        copy1 = pltpu.make_async_copy(
            p_ref.at[pl.ds(i * rows, rows), :], acc.at[s], copy1_sem
        )
        copy1.start()
        copy1.wait()
        d1 = rdma1(i)
        d1.wait_send()
        d1.wait_recv()
        acc[s] = acc[s] + recv[i % NSLOT]

        d2 = rdma2(i)
        d2.start()
        pltpu.make_async_copy(
            acc.at[s], out_ref.at[pl.ds(out0 + i * rows, rows), :],
            copy2_sem.at[s],
        ).start()
        d2.wait_recv()

    for i in range(NCHUNK - 2, NCHUNK):
        rdma2(i).wait_send()
        pltpu.make_async_copy(
            acc.at[i % 2], out_ref.at[pl.ds(out0 + i * rows, rows), :],
            copy2_sem.at[i % 2],
        ).wait()


def _allreduce_x(partial, my_y):
    half, n = partial.shape
    rows = half // NCHUNK
    m = 2 * half
    return pl.pallas_call(
        _allreduce_x_body,
        out_shape=jax.ShapeDtypeStruct((m, n), partial.dtype),
        in_specs=[pl.BlockSpec(memory_space=pl.ANY)],
        out_specs=pl.BlockSpec(memory_space=pl.ANY),
        scratch_shapes=[
            pltpu.VMEM((2, rows, n), partial.dtype),
            pltpu.VMEM((NSLOT, rows, n), partial.dtype),
            pltpu.SemaphoreType.DMA((NSLOT,)),
            pltpu.SemaphoreType.DMA((NSLOT,)),
            pltpu.SemaphoreType.DMA((NSLOT,)),
            pltpu.SemaphoreType.DMA((NSLOT,)),
            pltpu.SemaphoreType.DMA,
            pltpu.SemaphoreType.DMA((2,)),
        ],
        compiler_params=pltpu.CompilerParams(collective_id=0),
    )(partial)


def kernel(dy, W):
    my_y = lax.axis_index("y")
    half = dy.shape[0] // 2
    dy_half = lax.dynamic_slice_in_dim(dy, my_y * half, half, axis=0)
    partial = lax.dot_general(
        dy_half, W,
        dimension_numbers=(((1,), (1,)), ((), ())),
        preferred_element_type=jnp.float32,
    )
    return _allreduce_x(partial, my_y)


# device time: 667453 ns/iter; 1.7330x vs baseline; 1.0467x over previous
import jax
import jax.numpy as jnp
from jax import lax
from jax.experimental import pallas as pl
from jax.experimental.pallas import tpu as pltpu

NCHUNK = 8


def _allreduce_x_body(
    p_ref, out_ref, acc, recv, send_sem1, recv_sem1, send_sem2, recv_sem2,
    copy1_sem, copy2_sem,
):
    _, rows, n = recv.shape
    half = NCHUNK * rows
    my_x = lax.axis_index("x")
    my_y = lax.axis_index("y")
    my_z = lax.axis_index("z")
    x_peer = (1 - my_x, my_y, my_z)
    y_peer = (my_x, 1 - my_y, my_z)

    barrier = pltpu.get_barrier_semaphore()
    for nbr in (x_peer, y_peer):
        pl.semaphore_signal(
            barrier, inc=1, device_id=nbr, device_id_type=pl.DeviceIdType.MESH
        )
    pl.semaphore_wait(barrier, 2)

    out0 = my_y * half

    def rdma1(i):
        return pltpu.make_async_remote_copy(
            src_ref=p_ref.at[pl.ds(i * rows, rows), :],
            dst_ref=recv.at[i],
            send_sem=send_sem1.at[i],
            recv_sem=recv_sem1.at[i],
            device_id=x_peer,
            device_id_type=pl.DeviceIdType.MESH,
        )

    def rdma2(i):
        return pltpu.make_async_remote_copy(
            src_ref=acc.at[i % 2],
            dst_ref=out_ref.at[pl.ds(out0 + i * rows, rows), :],
            send_sem=send_sem2.at[i],
            recv_sem=recv_sem2.at[i],
            device_id=y_peer,
            device_id_type=pl.DeviceIdType.MESH,
        )

    def copy2(i):
        return pltpu.make_async_copy(
            acc.at[i % 2], out_ref.at[pl.ds(out0 + i * rows, rows), :],
            copy2_sem.at[i % 2],
        )

    for i in range(NCHUNK):
        rdma1(i).start()

    for i in range(NCHUNK):
        s = i % 2
        if i >= 2:
            rdma2(i - 2).wait_send()
            copy2(i - 2).wait()
        copy1 = pltpu.make_async_copy(
            p_ref.at[pl.ds(i * rows, rows), :], acc.at[s], copy1_sem
        )
        copy1.start()
        copy1.wait()
        d1 = rdma1(i)
        d1.wait_send()
        d1.wait_recv()
        acc[s] = acc[s] + recv[i]
        rdma2(i).start()
        copy2(i).start()

    for i in range(NCHUNK - 2, NCHUNK):
        rdma2(i).wait_send()
        copy2(i).wait()
    for i in range(NCHUNK):
        rdma2(i).wait_recv()


def _allreduce_x(partial):
    half, n = partial.shape
    rows = half // NCHUNK
    m = 2 * half
    return pl.pallas_call(
        _allreduce_x_body,
        out_shape=jax.ShapeDtypeStruct((m, n), partial.dtype),
        in_specs=[pl.BlockSpec(memory_space=pl.ANY)],
        out_specs=pl.BlockSpec(memory_space=pl.ANY),
        scratch_shapes=[
            pltpu.VMEM((2, rows, n), partial.dtype),
            pltpu.VMEM((NCHUNK, rows, n), partial.dtype),
            pltpu.SemaphoreType.DMA((NCHUNK,)),
            pltpu.SemaphoreType.DMA((NCHUNK,)),
            pltpu.SemaphoreType.DMA((NCHUNK,)),
            pltpu.SemaphoreType.DMA((NCHUNK,)),
            pltpu.SemaphoreType.DMA,
            pltpu.SemaphoreType.DMA((2,)),
        ],
        compiler_params=pltpu.CompilerParams(
            collective_id=0, vmem_limit_bytes=60 * 1024 * 1024
        ),
    )(partial)


def kernel(dy, W):
    my_y = lax.axis_index("y")
    half = dy.shape[0] // 2
    dy_half = lax.dynamic_slice_in_dim(dy, my_y * half, half, axis=0)
    partial = lax.dot_general(
        dy_half, W,
        dimension_numbers=(((1,), (1,)), ((), ())),
        preferred_element_type=jnp.float32,
    )
    return _allreduce_x(partial)
